# baseline (speedup 1.0000x reference)
import functools

import jax
import jax.numpy as jnp
from jax import lax
from jax.experimental import pallas as pl
from jax.experimental.pallas import tpu as pltpu
from jax.experimental.pallas import tpu_sc as plsc

VOCAB = 100000
EMB = 64
NPROP = 26
BATCH = 16384

NC = 2
NS = 16
NW = NC * NS
B_W = BATCH // NW
ICHUNK = 128
NJ = B_W // ICHUNK
IDXN = NPROP * B_W
LANES = 16
PAIR = 2 * EMB
GROUP = LANES


def _fire(table_hbm, idx_v, buf, sem, k):
    pltpu.async_copy(
        table_hbm.at[idx_v.at[pl.ds(k * ICHUNK, ICHUNK)]],
        buf,
        sem,
    )


def _drain(table_hbm, buf, sem):
    pltpu.make_async_copy(table_hbm.at[pl.ds(0, ICHUNK)], buf, sem).wait()


def _accumulate(acc_v, par_v, buf, p, j):
    @plsc.parallel_loop(0, ICHUNK // GROUP, unroll=1)
    def _(g):
        row0 = g * GROUP
        pvg = par_v[pl.ds(p * B_W + j * ICHUNK + row0, LANES)]
        for rloc in range(GROUP):
            pr = jnp.take_along_axis(
                pvg, jnp.full((LANES,), rloc, jnp.int32), axis=0)
            row = row0 + rloc
            acc_row = j * (ICHUNK // 2) + g * (GROUP // 2) + rloc // 2
            half = (rloc & 1) * EMB
            for c in range(EMB // LANES):
                xlo = buf[row, pl.ds(c * LANES, LANES)]
                xhi = buf[row, pl.ds(EMB + c * LANES, LANES)]
                x = xlo + pr * (xhi - xlo)
                plsc.addupdate(
                    acc_v.at[acc_row, pl.ds(half + c * LANES, LANES)], x)


def _emb_body(propw_hbm, table_hbm, out_hbm, idx_v, par_v, bufs, acc_v, sems):
    wid = lax.axis_index("s") * NC + lax.axis_index("c")

    pltpu.sync_copy(propw_hbm.at[pl.ds(wid * IDXN, IDXN)], idx_v)

    @plsc.parallel_loop(0, IDXN // LANES, unroll=4)
    def _(k):
        off = (k // (B_W // LANES)) * jnp.int32(VOCAB)
        sl = pl.ds(k * LANES, LANES)
        g = idx_v[sl] + off
        idx_v[sl] = lax.shift_right_logical(g, 1)
        par_v[sl] = lax.bitwise_and(g, 1).astype(jnp.float32)

    zeros = jnp.zeros((LANES,), jnp.float32)

    @plsc.parallel_loop(0, B_W // 2, unroll=4)
    def _(i):
        for c in range(PAIR // LANES):
            acc_v[i, pl.ds(c * LANES, LANES)] = zeros

    for j in range(NJ):
        _fire(table_hbm, idx_v, bufs[j], sems[j], jnp.int32(j))

    def p_step(p, _):
        for j in range(NJ):
            _drain(table_hbm, bufs[j], sems[j])
            _accumulate(acc_v, par_v, bufs[j], p, j)

            @pl.when(p + 1 < NPROP)
            def _():
                _fire(table_hbm, idx_v, bufs[j], sems[j], (p + 1) * NJ + j)

        return 0

    lax.fori_loop(0, NPROP, p_step, 0)

    pltpu.sync_copy(acc_v, out_hbm.at[pl.ds(wid * (B_W // 2), B_W // 2)])


def _body_wrap(propw_hbm, table_hbm, out_hbm, idx_v, par_v,
               b0, b1, b2, b3, acc_v, s0, s1, s2, s3):
    _emb_body(propw_hbm, table_hbm, out_hbm, idx_v, par_v,
              [b0, b1, b2, b3], acc_v, [s0, s1, s2, s3])


@jax.jit
def _emb_call(propw, pair_table):
    mesh = plsc.VectorSubcoreMesh(core_axis_name="c", subcore_axis_name="s")
    f = functools.partial(
        pl.kernel,
        out_type=jax.ShapeDtypeStruct((BATCH // 2, PAIR), jnp.float32),
        mesh=mesh,
        scratch_types=[
            pltpu.VMEM((IDXN,), jnp.int32),
            pltpu.VMEM((IDXN,), jnp.float32),
            pltpu.VMEM((ICHUNK, PAIR), jnp.float32),
            pltpu.VMEM((ICHUNK, PAIR), jnp.float32),
            pltpu.VMEM((ICHUNK, PAIR), jnp.float32),
            pltpu.VMEM((ICHUNK, PAIR), jnp.float32),
            pltpu.VMEM((B_W // 2, PAIR), jnp.float32),
            pltpu.SemaphoreType.DMA,
            pltpu.SemaphoreType.DMA,
            pltpu.SemaphoreType.DMA,
            pltpu.SemaphoreType.DMA,
        ],
    )(_body_wrap)
    return f(propw, pair_table)


def kernel(prop, tables):
    propw = (
        prop.astype(jnp.int32)
        .reshape(NW, B_W, NPROP)
        .transpose(0, 2, 1)
        .reshape(-1)
    )
    pair_table = tables.reshape(NPROP * VOCAB // 2, PAIR)
    out = _emb_call(propw, pair_table)
    return out.reshape(BATCH, 1, EMB)


# speedup vs baseline: 1.0556x; 1.0556x over previous
import functools

import jax
import jax.numpy as jnp
from jax import lax
from jax.experimental import pallas as pl
from jax.experimental.pallas import tpu as pltpu
from jax.experimental.pallas import tpu_sc as plsc

VOCAB = 100000
EMB = 64
NPROP = 26
BATCH = 16384

NC = 2
NS = 16
NW = NC * NS
B_W = BATCH // NW
IDXN = NPROP * B_W
ROWS_C = 4
CHUNK = ROWS_C * NPROP
NCHUNK = B_W // ROWS_C
NBUF = 4
LANES = 16


def _fire(table_hbm, idx_v, buf, sem, k):
    pltpu.async_copy(
        table_hbm.at[idx_v.at[pl.ds(k * CHUNK, CHUNK)]],
        buf,
        sem,
    )


def _drain(table_hbm, buf, sem):
    pltpu.make_async_copy(table_hbm.at[pl.ds(0, CHUNK)], buf, sem).wait()


def _accumulate(acc_v, buf, k):
    for b in range(ROWS_C):
        for c in range(EMB // LANES):
            sl = pl.ds(c * LANES, LANES)
            s = buf[b * NPROP, sl]
            for r in range(1, NPROP):
                s = s + buf[b * NPROP + r, sl]
            acc_v[ROWS_C * k + b, sl] = s


def _emb_body(propf_hbm, table_hbm, out_hbm, idx_v, bufs, acc_v, sems):
    wid = lax.axis_index("s") * NC + lax.axis_index("c")

    pltpu.sync_copy(propf_hbm.at[pl.ds(wid * IDXN, IDXN)], idx_v)

    iota = lax.iota(jnp.int32, LANES)

    @plsc.parallel_loop(0, IDXN // LANES, unroll=4)
    def _(kk):
        pos = kk * LANES + iota
        off = lax.rem(pos, jnp.int32(NPROP)) * jnp.int32(VOCAB)
        sl = pl.ds(kk * LANES, LANES)
        idx_v[sl] = idx_v[sl] + off

    for s in range(NBUF):
        _fire(table_hbm, idx_v, bufs[s], sems[s], jnp.int32(s))

    def kk_step(kk, _):
        for s in range(NBUF):
            k = NBUF * kk + s
            _drain(table_hbm, bufs[s], sems[s])
            _accumulate(acc_v, bufs[s], k)

            @pl.when(kk < NCHUNK // NBUF - 1)
            def _():
                _fire(table_hbm, idx_v, bufs[s], sems[s], k + NBUF)

        return 0

    lax.fori_loop(0, NCHUNK // NBUF, kk_step, 0)

    pltpu.sync_copy(acc_v, out_hbm.at[pl.ds(wid * B_W, B_W)])


def _body_wrap(propf_hbm, table_hbm, out_hbm, idx_v,
               b0, b1, b2, b3, acc_v, s0, s1, s2, s3):
    _emb_body(propf_hbm, table_hbm, out_hbm, idx_v,
              [b0, b1, b2, b3], acc_v, [s0, s1, s2, s3])


@jax.jit
def _emb_call(propf, flat_table):
    mesh = plsc.VectorSubcoreMesh(core_axis_name="c", subcore_axis_name="s")
    f = functools.partial(
        pl.kernel,
        out_type=jax.ShapeDtypeStruct((BATCH, EMB), jnp.float32),
        mesh=mesh,
        scratch_types=[
            pltpu.VMEM((IDXN,), jnp.int32),
            pltpu.VMEM((CHUNK, EMB), jnp.float32),
            pltpu.VMEM((CHUNK, EMB), jnp.float32),
            pltpu.VMEM((CHUNK, EMB), jnp.float32),
            pltpu.VMEM((CHUNK, EMB), jnp.float32),
            pltpu.VMEM((B_W, EMB), jnp.float32),
            pltpu.SemaphoreType.DMA,
            pltpu.SemaphoreType.DMA,
            pltpu.SemaphoreType.DMA,
            pltpu.SemaphoreType.DMA,
        ],
        compiler_params=pltpu.CompilerParams(use_tc_tiling_on_sc=False),
    )(_body_wrap)
    return f(propf, flat_table)


def kernel(prop, tables):
    propf = prop.astype(jnp.int32).reshape(-1)
    flat_table = tables.reshape(NPROP * VOCAB, EMB)
    out = _emb_call(propf, flat_table)
    return out[:, None, :]


# speedup vs baseline: 1.1334x; 1.0737x over previous
import functools

import jax
import jax.numpy as jnp
from jax import lax
from jax.experimental import pallas as pl
from jax.experimental.pallas import tpu as pltpu
from jax.experimental.pallas import tpu_sc as plsc

VOCAB = 100000
EMB = 64
NPROP = 26
BATCH = 16384

NC = 2
NS = 16
NW = NC * NS
B_W = BATCH // NW
ICHUNK = 128
NJ = B_W // ICHUNK
IDXN = NPROP * B_W
LANES = 16


def _fire(table_hbm, idx_v, buf, sem, p):
    for j in range(NJ):
        pltpu.async_copy(
            table_hbm.at[idx_v.at[pl.ds((p * NJ + j) * ICHUNK, ICHUNK)]],
            buf.at[pl.ds(j * ICHUNK, ICHUNK)],
            sem,
        )


def _drain(table_hbm, buf, sem):
    pltpu.make_async_copy(table_hbm.at[pl.ds(0, B_W)], buf, sem).wait()


def _accumulate(acc_v, buf):
    @plsc.parallel_loop(0, B_W, unroll=8)
    def _(i):
        for c in range(EMB // LANES):
            sl = pl.ds(c * LANES, LANES)
            plsc.addupdate(acc_v.at[i, sl], buf[i, sl])


def _emb_body(propw_hbm, table_hbm, out_hbm, idx_v, buf_a, buf_b, acc_v,
              sem_a, sem_b):
    wid = lax.axis_index("s") * NC + lax.axis_index("c")
    base = wid * B_W

    pltpu.sync_copy(propw_hbm.at[pl.ds(wid * IDXN, IDXN)], idx_v)

    @plsc.parallel_loop(0, IDXN // LANES, unroll=4)
    def _(k):
        off = (k // (B_W // LANES)) * jnp.int32(VOCAB)
        sl = pl.ds(k * LANES, LANES)
        idx_v[sl] = idx_v[sl] + off

    zeros = jnp.zeros((LANES,), jnp.float32)

    @plsc.parallel_loop(0, B_W, unroll=8)
    def _(i):
        for c in range(EMB // LANES):
            acc_v[i, pl.ds(c * LANES, LANES)] = zeros

    _fire(table_hbm, idx_v, buf_a, sem_a, jnp.int32(0))

    def pp_step(pp, _):
        p = 2 * pp
        _fire(table_hbm, idx_v, buf_b, sem_b, p + 1)
        _drain(table_hbm, buf_a, sem_a)
        _accumulate(acc_v, buf_a)

        @pl.when(pp < NPROP // 2 - 1)
        def _():
            _fire(table_hbm, idx_v, buf_a, sem_a, p + 2)

        _drain(table_hbm, buf_b, sem_b)
        _accumulate(acc_v, buf_b)
        return 0

    lax.fori_loop(0, NPROP // 2, pp_step, 0)

    pltpu.sync_copy(acc_v, out_hbm.at[pl.ds(base, B_W)])


@jax.jit
def _emb_call(propw, flat_table):
    mesh = plsc.VectorSubcoreMesh(core_axis_name="c", subcore_axis_name="s")
    f = functools.partial(
        pl.kernel,
        out_type=jax.ShapeDtypeStruct((BATCH, EMB), jnp.float32),
        mesh=mesh,
        scratch_types=[
            pltpu.VMEM((IDXN,), jnp.int32),
            pltpu.VMEM((B_W, EMB), jnp.float32),
            pltpu.VMEM((B_W, EMB), jnp.float32),
            pltpu.VMEM((B_W, EMB), jnp.float32),
            pltpu.SemaphoreType.DMA,
            pltpu.SemaphoreType.DMA,
        ],
        compiler_params=pltpu.CompilerParams(use_tc_tiling_on_sc=False),
    )(_emb_body)
    return f(propw, flat_table)


def kernel(prop, tables):
    propw = (
        prop.astype(jnp.int32)
        .reshape(NW, B_W, NPROP)
        .transpose(0, 2, 1)
        .reshape(-1)
    )
    flat_table = tables.reshape(NPROP * VOCAB, EMB)
    out = _emb_call(propw, flat_table)
    return out[:, None, :]
